# baseline (speedup 1.0000x reference)
import functools
import jax
import jax.numpy as jnp
from jax import lax
from jax.experimental import pallas as pl
from jax.experimental.pallas import tpu as pltpu
from jax.experimental.pallas import tpu_sc as plsc

N = 10000
E = 320000
D = 128
SG = 16
HL = 128

_NPAD = 10240


def _leaky(x):
    return jnp.where(x > 0, x, 0.01 * x)


def _mm_body(x_ref, w_ref, o_ref, *, act):
    y = jnp.dot(x_ref[...], w_ref[...], preferred_element_type=jnp.float32)
    if act:
        y = _leaky(y)
    o_ref[...] = y


def _mm_res_body(x_ref, w_ref, r_ref, o_ref, *, act):
    y = r_ref[...] + jnp.dot(x_ref[...], w_ref[...], preferred_element_type=jnp.float32)
    if act:
        y = _leaky(y)
    o_ref[...] = y


def _mm(x, w, res=None, act=True, br=1024):
    rows = x.shape[0]
    assert rows % br == 0, (rows, br)
    k = x.shape[1]
    n = w.shape[1]
    grid = rows // br
    in_specs = [
        pl.BlockSpec((br, k), lambda i: (i, 0)),
        pl.BlockSpec((k, n), lambda i: (0, 0)),
    ]
    args = [x, w]
    if res is not None:
        in_specs.append(pl.BlockSpec((br, n), lambda i: (i, 0)))
        args.append(res)
        body = functools.partial(_mm_res_body, act=act)
    else:
        body = functools.partial(_mm_body, act=act)
    return pl.pallas_call(
        body,
        grid=(grid,),
        in_specs=in_specs,
        out_specs=pl.BlockSpec((br, n), lambda i: (i, 0)),
        out_shape=jax.ShapeDtypeStruct((rows, n), jnp.float32),
    )(*args)


def _upd_body(h_ref, a_ref, w_ref, o_ref):
    o_ref[...] = _leaky(h_ref[...] + jnp.dot(a_ref[...], w_ref[...],
                                             preferred_element_type=jnp.float32))


def _update(h, agg, w, br=1024):
    rows = h.shape[0]
    grid = rows // br
    return pl.pallas_call(
        _upd_body,
        grid=(grid,),
        in_specs=[
            pl.BlockSpec((br, D), lambda i: (i, 0)),
            pl.BlockSpec((br, D), lambda i: (i, 0)),
            pl.BlockSpec((D, D), lambda i: (0, 0)),
        ],
        out_specs=pl.BlockSpec((br, D), lambda i: (i, 0)),
        out_shape=jax.ShapeDtypeStruct((rows, D), jnp.float32),
    )(h, agg, w)


_EBR = 2560


def _eo_body(acc_ref, e8_ref, gt_ref, we_ref, ws_ref,
             eo_ref, loss_ref):
    i = pl.program_id(0)
    eo = _leaky(acc_ref[...] + jnp.dot(e8_ref[...], we_ref[...],
                                       preferred_element_type=jnp.float32))
    eo_ref[...] = eo
    logit = jnp.dot(eo, ws_ref[...], preferred_element_type=jnp.float32)[:, 0]
    s = jax.nn.sigmoid(logit)
    d = s - gt_ref[0, 0, :]
    part = jnp.sum(d * d)

    @pl.when(i == 0)
    def _():
        loss_ref[...] = jnp.zeros_like(loss_ref)

    loss_ref[...] += jnp.full((1, 1), 0.0, jnp.float32) + part


def _eo_head(eoacc, e8, gt3, we_e8, ws):
    grid = E // _EBR
    return pl.pallas_call(
        _eo_body,
        grid=(grid,),
        in_specs=[
            pl.BlockSpec((_EBR, D), lambda i: (i, 0)),
            pl.BlockSpec((_EBR, 8), lambda i: (i, 0)),
            pl.BlockSpec((1, 1, _EBR), lambda i: (i, 0, 0)),
            pl.BlockSpec((8, D), lambda i: (0, 0)),
            pl.BlockSpec((D, 1), lambda i: (0, 0)),
        ],
        out_specs=[
            pl.BlockSpec((_EBR, D), lambda i: (i, 0)),
            pl.BlockSpec((1, 1), lambda i: (0, 0)),
        ],
        out_shape=[
            jax.ShapeDtypeStruct((E, D), jnp.float32),
            jax.ShapeDtypeStruct((1, 1), jnp.float32),
        ],
    )(eoacc, e8, gt3, we_e8, ws)


def _vmean_body(sub_ref, agg_ref, wg_ref, vm_ref, s1_ref, s2_ref):
    i = pl.program_id(0)
    rows = _leaky(sub_ref[...] + jnp.dot(agg_ref[...], wg_ref[...],
                                         preferred_element_type=jnp.float32))
    vm = jnp.mean(rows.reshape(-1, SG, D), axis=1)
    vm_ref[...] = vm

    @pl.when(i == 0)
    def _():
        s1_ref[...] = jnp.zeros_like(s1_ref)
        s2_ref[...] = jnp.zeros_like(s2_ref)

    s1_ref[...] += jnp.sum(vm, axis=0, keepdims=True)
    s2_ref[...] += jnp.sum(vm * vm, axis=0, keepdims=True)


def _vmean_stats(sub, agg2, wg):
    grid = E // _EBR
    gb = _EBR // SG
    return pl.pallas_call(
        _vmean_body,
        grid=(grid,),
        in_specs=[
            pl.BlockSpec((_EBR, D), lambda i: (i, 0)),
            pl.BlockSpec((_EBR, D), lambda i: (i, 0)),
            pl.BlockSpec((D, D), lambda i: (0, 0)),
        ],
        out_specs=[
            pl.BlockSpec((gb, D), lambda i: (i, 0)),
            pl.BlockSpec((1, D), lambda i: (0, 0)),
            pl.BlockSpec((1, D), lambda i: (0, 0)),
        ],
        out_shape=[
            jax.ShapeDtypeStruct((E // SG, D), jnp.float32),
            jax.ShapeDtypeStruct((1, D), jnp.float32),
            jax.ShapeDtypeStruct((1, D), jnp.float32),
        ],
    )(sub, agg2, wg)


def _bnmm_body(x_ref, s1_ref, s2_ref, g_ref, b_ref, l_ref, bl_ref,
               y_ref, t1_ref, t2_ref, *, m, stats):
    i = pl.program_id(0)
    mu = s1_ref[...] / m
    var = s2_ref[...] / m - mu * mu
    rstd = jax.lax.rsqrt(var + 1e-5)
    xn = _leaky((x_ref[...] - mu) * rstd * g_ref[...] + b_ref[...])
    y = jnp.dot(xn, l_ref[...], preferred_element_type=jnp.float32) + bl_ref[...]
    y_ref[...] = y
    if stats:
        @pl.when(i == 0)
        def _():
            t1_ref[...] = jnp.zeros_like(t1_ref)
            t2_ref[...] = jnp.zeros_like(t2_ref)

        t1_ref[...] += jnp.sum(y, axis=0, keepdims=True)
        t2_ref[...] += jnp.sum(y * y, axis=0, keepdims=True)


def _bn_mm(x, s1, s2, g, b, L, bL, stats=True, br=2000):
    rows, k = x.shape
    n = L.shape[1]
    grid = rows // br
    outs = [jax.ShapeDtypeStruct((rows, n), jnp.float32)]
    out_specs = [pl.BlockSpec((br, n), lambda i: (i, 0))]
    if stats:
        outs += [jax.ShapeDtypeStruct((1, n), jnp.float32)] * 2
        out_specs += [pl.BlockSpec((1, n), lambda i: (0, 0))] * 2
    else:
        outs += [jax.ShapeDtypeStruct((1, 1), jnp.float32)] * 2
        out_specs += [pl.BlockSpec((1, 1), lambda i: (0, 0))] * 2
    res = pl.pallas_call(
        functools.partial(_bnmm_body, m=float(rows), stats=stats),
        grid=(grid,),
        in_specs=[
            pl.BlockSpec((br, k), lambda i: (i, 0)),
            pl.BlockSpec((1, k), lambda i: (0, 0)),
            pl.BlockSpec((1, k), lambda i: (0, 0)),
            pl.BlockSpec((1, k), lambda i: (0, 0)),
            pl.BlockSpec((1, k), lambda i: (0, 0)),
            pl.BlockSpec((k, n), lambda i: (0, 0)),
            pl.BlockSpec((1, n), lambda i: (0, 0)),
        ],
        out_specs=out_specs,
        out_shape=outs,
    )(x, s1, s2, g, b, L, bL)
    return res


def _seg_global(sub, sep):
    agg = jax.ops.segment_sum(sub[sep[0]], sep[1], num_segments=E)
    agg = agg + jax.ops.segment_sum(sub[sep[1]], sep[0], num_segments=E)
    return agg


def _medge_body(hs_ref, e_ref, w_ref, o_ref):
    cat = jnp.concatenate([hs_ref[...], e_ref[...]], axis=-1)
    o_ref[...] = _leaky(jnp.dot(cat, w_ref[...], preferred_element_type=jnp.float32))


def _medge(hsrc, e4, wm):
    grid = E // _EBR
    return pl.pallas_call(
        _medge_body,
        grid=(grid,),
        in_specs=[
            pl.BlockSpec((_EBR, D), lambda i: (i, 0)),
            pl.BlockSpec((_EBR, 4), lambda i: (i, 0)),
            pl.BlockSpec((D + 4, D), lambda i: (0, 0)),
        ],
        out_specs=pl.BlockSpec((_EBR, D), lambda i: (i, 0)),
        out_shape=jax.ShapeDtypeStruct((E, D), jnp.float32),
    )(hsrc, e4, wm)


def _eof_body(hs_ref, hd_ref, e_ref, gt_ref, we_ref, ws_ref, eo_ref, loss_ref):
    i = pl.program_id(0)
    cat = jnp.concatenate([hs_ref[...], hd_ref[...], e_ref[...]], axis=-1)
    eo = _leaky(jnp.dot(cat, we_ref[...], preferred_element_type=jnp.float32))
    eo_ref[...] = eo
    logit = jnp.dot(eo, ws_ref[...], preferred_element_type=jnp.float32)[:, 0]
    s = jax.nn.sigmoid(logit)
    d = s - gt_ref[0, 0, :]
    part = jnp.sum(d * d)

    @pl.when(i == 0)
    def _():
        loss_ref[...] = jnp.zeros_like(loss_ref)

    loss_ref[...] += jnp.full((1, 1), 0.0, jnp.float32) + part


def _eo_fused(hsrc, hdst, e4, gt3, we, ws):
    grid = E // _EBR
    return pl.pallas_call(
        _eof_body,
        grid=(grid,),
        in_specs=[
            pl.BlockSpec((_EBR, D), lambda i: (i, 0)),
            pl.BlockSpec((_EBR, D), lambda i: (i, 0)),
            pl.BlockSpec((_EBR, 4), lambda i: (i, 0)),
            pl.BlockSpec((1, 1, _EBR), lambda i: (i, 0, 0)),
            pl.BlockSpec((2 * D + 4, D), lambda i: (0, 0)),
            pl.BlockSpec((D, 1), lambda i: (0, 0)),
        ],
        out_specs=[
            pl.BlockSpec((_EBR, D), lambda i: (i, 0)),
            pl.BlockSpec((1, 1), lambda i: (0, 0)),
        ],
        out_shape=[
            jax.ShapeDtypeStruct((E, D), jnp.float32),
            jax.ShapeDtypeStruct((1, 1), jnp.float32),
        ],
    )(hsrc, hdst, e4, gt3, we, ws)


_GWB = 400


def _sc_gather(table, idx):
    rows_total = idx.shape[0]
    per_w = rows_total // 32
    nb = per_w // _GWB
    mesh = plsc.VectorSubcoreMesh(core_axis_name="c", subcore_axis_name="s", num_cores=2, num_subcores=16)

    @functools.partial(
        pl.kernel,
        out_type=jax.ShapeDtypeStruct((rows_total, D), jnp.float32),
        mesh=mesh,
        scratch_types=[
            pltpu.VMEM((_GWB,), jnp.int32),
            pltpu.VMEM((_GWB, D), jnp.float32),
            pltpu.SemaphoreType.DMA,
        ],
        compiler_params=pltpu.CompilerParams(needs_layout_passes=False),
    )
    def k(table_hbm, idx_hbm, out_hbm, idx_v, rows_v, sem):
        w = lax.axis_index("c") * 16 + lax.axis_index("s")
        base = w * per_w

        def body(b, carry):
            off = base + b * _GWB
            pltpu.sync_copy(idx_hbm.at[pl.ds(off, _GWB)], idx_v)
            pltpu.async_copy(table_hbm.at[idx_v], rows_v, sem).wait()
            pltpu.sync_copy(rows_v, out_hbm.at[pl.ds(off, _GWB)])
            return carry

        lax.fori_loop(0, nb, body, 0)

    return k(table, idx)


_CH = 6400
_NCH = E // _CH
_GB = 64


def _sc_segsum(m, dst, zrows):
    half = _NPAD // 2
    trows = _NPAD // 32
    mesh = plsc.VectorSubcoreMesh(core_axis_name="c", subcore_axis_name="s", num_cores=2, num_subcores=16)

    @functools.partial(
        pl.kernel,
        out_type=jax.ShapeDtypeStruct((_NPAD, D), jnp.float32),
        mesh=mesh,
        scratch_types=[
            pltpu.VMEM((_CH,), jnp.int32),
            pltpu.VMEM((_CH + 128,), jnp.int32),
            pltpu.VMEM(((_CH + 128) // _GB, _GB), jnp.int32),
            pltpu.VMEM((_GB, D), jnp.float32),
            pltpu.SemaphoreType.DMA,
            pltpu.VMEM_SHARED((_NPAD // 2 + 8, D), jnp.float32),
        ],
        compiler_params=pltpu.CompilerParams(needs_layout_passes=False),
    )
    def k(m_hbm, dst_hbm, z_hbm, agg_hbm,
          dstc_v, idbuf, dlbuf, rows_v, sem, acc_sh):
        c = lax.axis_index("c")
        s = lax.axis_index("s")
        w = c * 16 + s
        lo = w * trows
        hi = lo + trows
        scbase = c * half

        pltpu.sync_copy(z_hbm.at[pl.ds(0, trows)],
                        acc_sh.at[pl.ds(s * trows, trows)])

        @pl.when(s == 0)
        def _():
            pltpu.sync_copy(z_hbm.at[pl.ds(0, 8)], acc_sh.at[pl.ds(half, 8)])

        def chunk_body(ci, carry):
            cbase = ci * _CH
            pltpu.sync_copy(dst_hbm.at[pl.ds(cbase, _CH)], dstc_v)

            lanes = lax.iota(jnp.int32, 16)
            trash_pos = jnp.full((16,), _CH + 64, jnp.int32) + lanes

            def grp(g, cnt):
                d = dstc_v[pl.ds(g * 16, 16)]
                eid = jnp.full((16,), cbase, jnp.int32) + g * 16 + lanes
                msk = (d >= lo) & (d < hi)
                ps = plsc.cumsum(jnp.where(msk, jnp.int32(1), jnp.int32(0)))
                pos = jnp.where(msk, cnt + ps - 1, trash_pos)
                plsc.store_scatter(idbuf, [pos], eid)
                plsc.store_scatter(dlbuf, [pos // _GB, pos % _GB], d - scbase)
                return cnt + ps[15]

            cnt = lax.fori_loop(0, _CH // 16, grp, jnp.int32(0))
            padi = jnp.zeros((16,), jnp.int32)
            padd = jnp.full((16,), half, jnp.int32)
            for t in range(4):
                pos = jnp.full((16,), cnt + t * 16, jnp.int32) + lanes
                plsc.store_scatter(idbuf, [pos], padi)
                plsc.store_scatter(dlbuf, [pos // _GB, pos % _GB], padd)
            nbat = (cnt + (_GB - 1)) // _GB

            def fl(j, carry2):
                pltpu.async_copy(m_hbm.at[idbuf.at[pl.ds(j * _GB, _GB)]], rows_v, sem).wait()
                pltpu.sync_copy(rows_v, acc_sh.at[dlbuf.at[j]], add=True)
                return carry2

            lax.fori_loop(0, nbat, fl, 0)
            return carry

        lax.fori_loop(0, _NCH, chunk_body, 0)
        pltpu.sync_copy(acc_sh.at[pl.ds(s * trows, trows)],
                        agg_hbm.at[pl.ds(lo, trows)])

    return k(m, dst, zrows)


def kernel(node_features, actions, edge_index, angles, sub_graphs, sep_subgraphs,
           gt_edges, post_data, params):
    src, dst = edge_index[0], edge_index[1]
    e4 = jnp.concatenate([actions, angles], axis=-1)
    gt3 = gt_edges.reshape(E // _EBR, 1, _EBR)
    x = jnp.pad(node_features, ((0, _NPAD - N), (0, 0)))
    sidx = sub_graphs[0]
    sep = sep_subgraphs[0]
    zrows = jnp.zeros((_NPAD // 32, D), jnp.float32)

    def branch(q):
        h = _mm(x, q['Wn0'], act=True)
        for _ in range(3):
            hsrc = _sc_gather(h, src)
            m = _medge(hsrc, e4, q['Wm'])
            agg = _sc_segsum(m, dst, zrows)
            h = _update(h, agg, q['Wu'])
        hsrc = _sc_gather(h, src)
        hdst = _sc_gather(h, dst)
        eo, loss_sum = _eo_fused(hsrc, hdst, e4, gt3, q['We'], q['ws'])
        return eo, loss_sum[0, 0] / E

    def head(eo, gg, v):
        sub = _sc_gather(eo, sidx)
        agg2 = _seg_global(sub, sep)
        vm, s1, s2 = _vmean_stats(sub, agg2, gg['Wg'])
        y1, t1, t2 = _bn_mm(vm, s1, s2, v['g1'].reshape(1, -1), v['b1'].reshape(1, -1),
                            v['L1'], v['bL1'].reshape(1, -1), stats=True)
        y2, u1, u2 = _bn_mm(y1, t1, t2, v['g2'].reshape(1, -1), v['b2'].reshape(1, -1),
                            v['L2'], v['bL2'].reshape(1, -1), stats=True)
        y3, _, _ = _bn_mm(y2, u1, u2, v['g3'].reshape(1, -1), v['b3'].reshape(1, -1),
                          v['L3'], v['bL3'].reshape(1, -1), stats=False)
        return y3[:, 0]

    eo1, l1 = branch(params['q1'])
    eo2, l2 = branch(params['q2'])
    v1 = head(eo1, params['gg1'], params['v1'])
    v2 = head(eo2, params['gg2'], params['v2'])
    return (v1, v2, (l1 + l2) / 4)


# speedup vs baseline: 1.0033x; 1.0033x over previous
import functools
import jax
import jax.numpy as jnp
from jax import lax
from jax.experimental import pallas as pl
from jax.experimental.pallas import tpu as pltpu
from jax.experimental.pallas import tpu_sc as plsc

N = 10000
E = 320000
D = 128
SG = 16
HL = 128

_NPAD = 10240


def _leaky(x):
    return jnp.where(x > 0, x, 0.01 * x)


def _mm_body(x_ref, w_ref, o_ref, *, act):
    y = jnp.dot(x_ref[...], w_ref[...], preferred_element_type=jnp.float32)
    if act:
        y = _leaky(y)
    o_ref[...] = y


def _mm_res_body(x_ref, w_ref, r_ref, o_ref, *, act):
    y = r_ref[...] + jnp.dot(x_ref[...], w_ref[...], preferred_element_type=jnp.float32)
    if act:
        y = _leaky(y)
    o_ref[...] = y


def _mm(x, w, res=None, act=True, br=1024):
    rows = x.shape[0]
    assert rows % br == 0, (rows, br)
    k = x.shape[1]
    n = w.shape[1]
    grid = rows // br
    in_specs = [
        pl.BlockSpec((br, k), lambda i: (i, 0)),
        pl.BlockSpec((k, n), lambda i: (0, 0)),
    ]
    args = [x, w]
    if res is not None:
        in_specs.append(pl.BlockSpec((br, n), lambda i: (i, 0)))
        args.append(res)
        body = functools.partial(_mm_res_body, act=act)
    else:
        body = functools.partial(_mm_body, act=act)
    return pl.pallas_call(
        body,
        grid=(grid,),
        in_specs=in_specs,
        out_specs=pl.BlockSpec((br, n), lambda i: (i, 0)),
        out_shape=jax.ShapeDtypeStruct((rows, n), jnp.float32),
    )(*args)


def _upd_body(h_ref, a_ref, w_ref, o_ref):
    o_ref[...] = _leaky(h_ref[...] + jnp.dot(a_ref[...], w_ref[...],
                                             preferred_element_type=jnp.float32))


def _update(h, agg, w, br=1024):
    rows = h.shape[0]
    grid = rows // br
    return pl.pallas_call(
        _upd_body,
        grid=(grid,),
        in_specs=[
            pl.BlockSpec((br, D), lambda i: (i, 0)),
            pl.BlockSpec((br, D), lambda i: (i, 0)),
            pl.BlockSpec((D, D), lambda i: (0, 0)),
        ],
        out_specs=pl.BlockSpec((br, D), lambda i: (i, 0)),
        out_shape=jax.ShapeDtypeStruct((rows, D), jnp.float32),
    )(h, agg, w)


_EBR = 2560


def _eo_body(acc_ref, e8_ref, gt_ref, we_ref, ws_ref,
             eo_ref, loss_ref):
    i = pl.program_id(0)
    eo = _leaky(acc_ref[...] + jnp.dot(e8_ref[...], we_ref[...],
                                       preferred_element_type=jnp.float32))
    eo_ref[...] = eo
    logit = jnp.dot(eo, ws_ref[...], preferred_element_type=jnp.float32)[:, 0]
    s = jax.nn.sigmoid(logit)
    d = s - gt_ref[0, 0, :]
    part = jnp.sum(d * d)

    @pl.when(i == 0)
    def _():
        loss_ref[...] = jnp.zeros_like(loss_ref)

    loss_ref[...] += jnp.full((1, 1), 0.0, jnp.float32) + part


def _eo_head(eoacc, e8, gt3, we_e8, ws):
    grid = E // _EBR
    return pl.pallas_call(
        _eo_body,
        grid=(grid,),
        in_specs=[
            pl.BlockSpec((_EBR, D), lambda i: (i, 0)),
            pl.BlockSpec((_EBR, 8), lambda i: (i, 0)),
            pl.BlockSpec((1, 1, _EBR), lambda i: (i, 0, 0)),
            pl.BlockSpec((8, D), lambda i: (0, 0)),
            pl.BlockSpec((D, 1), lambda i: (0, 0)),
        ],
        out_specs=[
            pl.BlockSpec((_EBR, D), lambda i: (i, 0)),
            pl.BlockSpec((1, 1), lambda i: (0, 0)),
        ],
        out_shape=[
            jax.ShapeDtypeStruct((E, D), jnp.float32),
            jax.ShapeDtypeStruct((1, 1), jnp.float32),
        ],
    )(eoacc, e8, gt3, we_e8, ws)


def _vmean_body(sub_ref, agg_ref, wg_ref, vm_ref, s1_ref, s2_ref):
    i = pl.program_id(0)
    rows = _leaky(sub_ref[...] + jnp.dot(agg_ref[...], wg_ref[...],
                                         preferred_element_type=jnp.float32))
    vm = jnp.mean(rows.reshape(-1, SG, D), axis=1)
    vm_ref[...] = vm

    @pl.when(i == 0)
    def _():
        s1_ref[...] = jnp.zeros_like(s1_ref)
        s2_ref[...] = jnp.zeros_like(s2_ref)

    s1_ref[...] += jnp.sum(vm, axis=0, keepdims=True)
    s2_ref[...] += jnp.sum(vm * vm, axis=0, keepdims=True)


def _vmean_stats(sub, agg2, wg):
    grid = E // _EBR
    gb = _EBR // SG
    return pl.pallas_call(
        _vmean_body,
        grid=(grid,),
        in_specs=[
            pl.BlockSpec((_EBR, D), lambda i: (i, 0)),
            pl.BlockSpec((_EBR, D), lambda i: (i, 0)),
            pl.BlockSpec((D, D), lambda i: (0, 0)),
        ],
        out_specs=[
            pl.BlockSpec((gb, D), lambda i: (i, 0)),
            pl.BlockSpec((1, D), lambda i: (0, 0)),
            pl.BlockSpec((1, D), lambda i: (0, 0)),
        ],
        out_shape=[
            jax.ShapeDtypeStruct((E // SG, D), jnp.float32),
            jax.ShapeDtypeStruct((1, D), jnp.float32),
            jax.ShapeDtypeStruct((1, D), jnp.float32),
        ],
    )(sub, agg2, wg)


def _bnmm_body(x_ref, s1_ref, s2_ref, g_ref, b_ref, l_ref, bl_ref,
               y_ref, t1_ref, t2_ref, *, m, stats):
    i = pl.program_id(0)
    mu = s1_ref[...] / m
    var = s2_ref[...] / m - mu * mu
    rstd = jax.lax.rsqrt(var + 1e-5)
    xn = _leaky((x_ref[...] - mu) * rstd * g_ref[...] + b_ref[...])
    y = jnp.dot(xn, l_ref[...], preferred_element_type=jnp.float32) + bl_ref[...]
    y_ref[...] = y
    if stats:
        @pl.when(i == 0)
        def _():
            t1_ref[...] = jnp.zeros_like(t1_ref)
            t2_ref[...] = jnp.zeros_like(t2_ref)

        t1_ref[...] += jnp.sum(y, axis=0, keepdims=True)
        t2_ref[...] += jnp.sum(y * y, axis=0, keepdims=True)


def _bn_mm(x, s1, s2, g, b, L, bL, stats=True, br=2000):
    rows, k = x.shape
    n = L.shape[1]
    grid = rows // br
    outs = [jax.ShapeDtypeStruct((rows, n), jnp.float32)]
    out_specs = [pl.BlockSpec((br, n), lambda i: (i, 0))]
    if stats:
        outs += [jax.ShapeDtypeStruct((1, n), jnp.float32)] * 2
        out_specs += [pl.BlockSpec((1, n), lambda i: (0, 0))] * 2
    else:
        outs += [jax.ShapeDtypeStruct((1, 1), jnp.float32)] * 2
        out_specs += [pl.BlockSpec((1, 1), lambda i: (0, 0))] * 2
    res = pl.pallas_call(
        functools.partial(_bnmm_body, m=float(rows), stats=stats),
        grid=(grid,),
        in_specs=[
            pl.BlockSpec((br, k), lambda i: (i, 0)),
            pl.BlockSpec((1, k), lambda i: (0, 0)),
            pl.BlockSpec((1, k), lambda i: (0, 0)),
            pl.BlockSpec((1, k), lambda i: (0, 0)),
            pl.BlockSpec((1, k), lambda i: (0, 0)),
            pl.BlockSpec((k, n), lambda i: (0, 0)),
            pl.BlockSpec((1, n), lambda i: (0, 0)),
        ],
        out_specs=out_specs,
        out_shape=outs,
    )(x, s1, s2, g, b, L, bL)
    return res


def _seg_global(sub, sep):
    agg = jax.ops.segment_sum(sub[sep[0]], sep[1], num_segments=E)
    agg = agg + jax.ops.segment_sum(sub[sep[1]], sep[0], num_segments=E)
    return agg


def _medge_body(hs_ref, e_ref, w_ref, o_ref):
    cat = jnp.concatenate([hs_ref[...], e_ref[...]], axis=-1)
    o_ref[...] = _leaky(jnp.dot(cat, w_ref[...], preferred_element_type=jnp.float32))


def _medge(hsrc, e4, wm):
    grid = E // _EBR
    return pl.pallas_call(
        _medge_body,
        grid=(grid,),
        in_specs=[
            pl.BlockSpec((_EBR, D), lambda i: (i, 0)),
            pl.BlockSpec((_EBR, 4), lambda i: (i, 0)),
            pl.BlockSpec((D + 4, D), lambda i: (0, 0)),
        ],
        out_specs=pl.BlockSpec((_EBR, D), lambda i: (i, 0)),
        out_shape=jax.ShapeDtypeStruct((E, D), jnp.float32),
    )(hsrc, e4, wm)


def _eof_body(hs_ref, hd_ref, e_ref, gt_ref, we_ref, ws_ref, eo_ref, loss_ref):
    i = pl.program_id(0)
    cat = jnp.concatenate([hs_ref[...], hd_ref[...], e_ref[...]], axis=-1)
    eo = _leaky(jnp.dot(cat, we_ref[...], preferred_element_type=jnp.float32))
    eo_ref[...] = eo
    logit = jnp.dot(eo, ws_ref[...], preferred_element_type=jnp.float32)[:, 0]
    s = jax.nn.sigmoid(logit)
    d = s - gt_ref[0, 0, :]
    part = jnp.sum(d * d)

    @pl.when(i == 0)
    def _():
        loss_ref[...] = jnp.zeros_like(loss_ref)

    loss_ref[...] += jnp.full((1, 1), 0.0, jnp.float32) + part


def _eo_fused(hsrc, hdst, e4, gt3, we, ws):
    grid = E // _EBR
    return pl.pallas_call(
        _eof_body,
        grid=(grid,),
        in_specs=[
            pl.BlockSpec((_EBR, D), lambda i: (i, 0)),
            pl.BlockSpec((_EBR, D), lambda i: (i, 0)),
            pl.BlockSpec((_EBR, 4), lambda i: (i, 0)),
            pl.BlockSpec((1, 1, _EBR), lambda i: (i, 0, 0)),
            pl.BlockSpec((2 * D + 4, D), lambda i: (0, 0)),
            pl.BlockSpec((D, 1), lambda i: (0, 0)),
        ],
        out_specs=[
            pl.BlockSpec((_EBR, D), lambda i: (i, 0)),
            pl.BlockSpec((1, 1), lambda i: (0, 0)),
        ],
        out_shape=[
            jax.ShapeDtypeStruct((E, D), jnp.float32),
            jax.ShapeDtypeStruct((1, 1), jnp.float32),
        ],
    )(hsrc, hdst, e4, gt3, we, ws)


_GWB = 400


def _sc_gather(table, idx):
    rows_total = idx.shape[0]
    per_w = rows_total // 32
    nb = per_w // _GWB
    mesh = plsc.VectorSubcoreMesh(core_axis_name="c", subcore_axis_name="s", num_cores=2, num_subcores=16)

    @functools.partial(
        pl.kernel,
        out_type=jax.ShapeDtypeStruct((rows_total, D), jnp.float32),
        mesh=mesh,
        scratch_types=[
            pltpu.VMEM((per_w,), jnp.int32),
            pltpu.VMEM((2, _GWB, D), jnp.float32),
            pltpu.SemaphoreType.DMA,
            pltpu.SemaphoreType.DMA,
        ],
        compiler_params=pltpu.CompilerParams(needs_layout_passes=False),
    )
    def k(table_hbm, idx_hbm, out_hbm, idxall, rows_v, gsem, wsem):
        w = lax.axis_index("c") * 16 + lax.axis_index("s")
        base = w * per_w
        pltpu.sync_copy(idx_hbm.at[pl.ds(base, per_w)], idxall)
        pltpu.async_copy(table_hbm.at[idxall.at[pl.ds(0, _GWB)]],
                         rows_v.at[0], gsem)

        def body(b, carry):
            @pl.when(b >= 2)
            def _():
                pltpu.make_async_copy(
                    rows_v.at[b % 2],
                    out_hbm.at[pl.ds(base + (b - 2) * _GWB, _GWB)], wsem).wait()

            pltpu.async_copy(table_hbm.at[idxall.at[pl.ds(b * _GWB, _GWB)]],
                             rows_v.at[b % 2], gsem)
            pltpu.make_async_copy(table_hbm.at[idxall.at[pl.ds(0, _GWB)]],
                                  rows_v.at[(b + 1) % 2], gsem).wait()
            pltpu.async_copy(rows_v.at[(b + 1) % 2],
                             out_hbm.at[pl.ds(base + (b - 1) * _GWB, _GWB)], wsem)
            return carry

        lax.fori_loop(1, nb, body, 0)
        pltpu.make_async_copy(table_hbm.at[idxall.at[pl.ds(0, _GWB)]],
                              rows_v.at[(nb - 1) % 2], gsem).wait()
        pltpu.async_copy(rows_v.at[(nb - 1) % 2],
                         out_hbm.at[pl.ds(base + (nb - 1) * _GWB, _GWB)], wsem)
        pltpu.make_async_copy(
            rows_v.at[0], out_hbm.at[pl.ds(base, _GWB)], wsem).wait()
        pltpu.make_async_copy(
            rows_v.at[0], out_hbm.at[pl.ds(base, _GWB)], wsem).wait()

    return k(table, idx)


_CH = 6400
_NCH = E // _CH
_GB = 64


def _sc_segsum(m, dst, zrows):
    half = _NPAD // 2
    trows = _NPAD // 32
    mesh = plsc.VectorSubcoreMesh(core_axis_name="c", subcore_axis_name="s", num_cores=2, num_subcores=16)

    @functools.partial(
        pl.kernel,
        out_type=jax.ShapeDtypeStruct((_NPAD, D), jnp.float32),
        mesh=mesh,
        scratch_types=[
            pltpu.VMEM((_CH,), jnp.int32),
            pltpu.VMEM((_CH + 128,), jnp.int32),
            pltpu.VMEM(((_CH + 128) // _GB, _GB), jnp.int32),
            pltpu.VMEM((2, _GB, D), jnp.float32),
            pltpu.SemaphoreType.DMA,
            pltpu.VMEM_SHARED((_NPAD // 2 + 8, D), jnp.float32),
        ],
        compiler_params=pltpu.CompilerParams(needs_layout_passes=False),
    )
    def k(m_hbm, dst_hbm, z_hbm, agg_hbm,
          dstc_v, idbuf, dlbuf, rows_v, sem, acc_sh):
        c = lax.axis_index("c")
        s = lax.axis_index("s")
        w = c * 16 + s
        lo = w * trows
        hi = lo + trows
        scbase = c * half

        pltpu.sync_copy(z_hbm.at[pl.ds(0, trows)],
                        acc_sh.at[pl.ds(s * trows, trows)])

        @pl.when(s == 0)
        def _():
            pltpu.sync_copy(z_hbm.at[pl.ds(0, 8)], acc_sh.at[pl.ds(half, 8)])

        def chunk_body(ci, carry):
            cbase = ci * _CH
            pltpu.sync_copy(dst_hbm.at[pl.ds(cbase, _CH)], dstc_v)

            lanes = lax.iota(jnp.int32, 16)
            trash_pos = jnp.full((16,), _CH + 64, jnp.int32) + lanes

            def grp(g, cnt):
                d = dstc_v[pl.ds(g * 16, 16)]
                eid = jnp.full((16,), cbase, jnp.int32) + g * 16 + lanes
                msk = (d >= lo) & (d < hi)
                ps = plsc.cumsum(jnp.where(msk, jnp.int32(1), jnp.int32(0)))
                pos = jnp.where(msk, cnt + ps - 1, trash_pos)
                plsc.store_scatter(idbuf, [pos], eid)
                plsc.store_scatter(dlbuf, [pos // _GB, pos % _GB], d - scbase)
                return cnt + ps[15]

            cnt = lax.fori_loop(0, _CH // 16, grp, jnp.int32(0))
            padi = jnp.zeros((16,), jnp.int32)
            padd = jnp.full((16,), half, jnp.int32)
            for t in range(4):
                pos = jnp.full((16,), cnt + t * 16, jnp.int32) + lanes
                plsc.store_scatter(idbuf, [pos], padi)
                plsc.store_scatter(dlbuf, [pos // _GB, pos % _GB], padd)
            nbat = (cnt + (_GB - 1)) // _GB

            @pl.when(nbat > 0)
            def _():
                pltpu.async_copy(m_hbm.at[idbuf.at[pl.ds(0, _GB)]], rows_v.at[0], sem)

            def fl(j, carry2):
                @pl.when(j + 1 < nbat)
                def _():
                    pltpu.async_copy(m_hbm.at[idbuf.at[pl.ds((j + 1) * _GB, _GB)]],
                                     rows_v.at[(j + 1) % 2], sem)

                pltpu.make_async_copy(m_hbm.at[idbuf.at[pl.ds(0, _GB)]],
                                      rows_v.at[j % 2], sem).wait()
                pltpu.sync_copy(rows_v.at[j % 2], acc_sh.at[dlbuf.at[j]], add=True)
                return carry2

            lax.fori_loop(0, nbat, fl, 0)
            return carry

        lax.fori_loop(0, _NCH, chunk_body, 0)
        pltpu.sync_copy(acc_sh.at[pl.ds(s * trows, trows)],
                        agg_hbm.at[pl.ds(lo, trows)])

    return k(m, dst, zrows)


def kernel(node_features, actions, edge_index, angles, sub_graphs, sep_subgraphs,
           gt_edges, post_data, params):
    src, dst = edge_index[0], edge_index[1]
    e4 = jnp.concatenate([actions, angles], axis=-1)
    gt3 = gt_edges.reshape(E // _EBR, 1, _EBR)
    x = jnp.pad(node_features, ((0, _NPAD - N), (0, 0)))
    sidx = sub_graphs[0]
    sep = sep_subgraphs[0]
    zrows = jnp.zeros((_NPAD // 32, D), jnp.float32)

    def branch(q):
        h = _mm(x, q['Wn0'], act=True)
        for _ in range(3):
            hsrc = _sc_gather(h, src)
            m = _medge(hsrc, e4, q['Wm'])
            agg = _sc_segsum(m, dst, zrows)
            h = _update(h, agg, q['Wu'])
        hsrc = _sc_gather(h, src)
        hdst = _sc_gather(h, dst)
        eo, loss_sum = _eo_fused(hsrc, hdst, e4, gt3, q['We'], q['ws'])
        return eo, loss_sum[0, 0] / E

    def head(eo, gg, v):
        sub = _sc_gather(eo, sidx)
        agg2 = _seg_global(sub, sep)
        vm, s1, s2 = _vmean_stats(sub, agg2, gg['Wg'])
        y1, t1, t2 = _bn_mm(vm, s1, s2, v['g1'].reshape(1, -1), v['b1'].reshape(1, -1),
                            v['L1'], v['bL1'].reshape(1, -1), stats=True)
        y2, u1, u2 = _bn_mm(y1, t1, t2, v['g2'].reshape(1, -1), v['b2'].reshape(1, -1),
                            v['L2'], v['bL2'].reshape(1, -1), stats=True)
        y3, _, _ = _bn_mm(y2, u1, u2, v['g3'].reshape(1, -1), v['b3'].reshape(1, -1),
                          v['L3'], v['bL3'].reshape(1, -1), stats=False)
        return y3[:, 0]

    eo1, l1 = branch(params['q1'])
    eo2, l2 = branch(params['q2'])
    v1 = head(eo1, params['gg1'], params['v1'])
    v2 = head(eo2, params['gg2'], params['v2'])
    return (v1, v2, (l1 + l2) / 4)
